# baseline (speedup 1.0000x reference)
import jax
import jax.numpy as jnp
from jax import lax
from jax.experimental import pallas as pl
from jax.experimental.pallas import tpu as pltpu

_B, _N, _W, _CW = 16, 256, 100, 10
_E, _CH, _H, _TOPK = 64, 64, 150, 30
_FH, _RH, _OUT = 150, 150, 256
_ALPHA = 0.2
_THRES = 0.1
_DIN = 2 * _W + _E + _CH


def _dot(a, b, dims):
    return lax.dot_general(a, b, (dims, ((), ())),
                           preferred_element_type=jnp.float32)


def _nn(a, b):
    return _dot(a, b, ((1,), (0,)))


def _nt(a, b):
    return _dot(a, b, ((1,), (1,)))


def _leaky(v):
    return jnp.where(v > 0, v, _ALPHA * v)


def _sig(v):
    return 0.5 * jnp.tanh(0.5 * v) + 0.5


def _graph_body(emb_ref, embt_ref, wfl_ref, sel_ref, wsel_ref, cos_ref):
    emb = emb_ref[...]
    embt = embt_ref[...]
    ncol = jnp.sqrt(jnp.sum(emb * emb, axis=1, keepdims=True))
    nrow = jnp.sqrt(jnp.sum(embt * embt, axis=0, keepdims=True))
    cos = _nt(emb, emb) / (ncol * nrow)
    ri = lax.broadcasted_iota(jnp.int32, (_N, _N), 0)
    ci = lax.broadcasted_iota(jnp.int32, (_N, _N), 1)
    cos = jnp.where(ri != ci, cos, 0.0)
    cos_ref[...] = cos

    work = cos
    sel = jnp.zeros((_N, _N), jnp.float32)
    wsel = jnp.zeros((_N, _N), jnp.float32)
    for k in range(_TOPK):
        m = jnp.max(work, axis=1, keepdims=True)
        jmin = jnp.min(jnp.where(work == m, ci, _N), axis=1, keepdims=True)
        oh = (ci == jmin).astype(jnp.float32)
        sel = sel + oh
        wsel = wsel + oh * wfl_ref[k, 0]
        work = work - oh * jnp.float32(1e30)
    sel_ref[...] = sel
    wsel_ref[...] = wsel


def _branch_body(x_ref, y_ref, emb_ref, wx_ref, a1_ref, a2_ref, bfl_ref,
                 w1_ref, w2_ref, wc_ref, bc_ref, sel_ref, wsel_ref, cos_ref,
                 out_ref):
    xb = x_ref[0]
    yb = y_ref[0]

    nr = _nn(xb, wx_ref[...]) + emb_ref[...]
    d1 = _nn(nr, a1_ref[...])
    d2r = _dot(a2_ref[...], nr, ((0,), (1,)))
    e = _leaky(d1 + d2r) + cos_ref[...]
    sel = sel_ref[...]
    em = jnp.where(sel > 0, e, -1e30)
    ex = jnp.exp(em - jnp.max(em, axis=1, keepdims=True)) * sel
    aw = ex / jnp.sum(ex, axis=1, keepdims=True) * wsel_ref[...]
    h_feat = _nn(aw, nr) + bfl_ref[0, 0]

    s1r = _dot(w1_ref[...], xb, ((0,), (0,)))
    s2c = _dot(xb, w2_ref[...], ((0,), (0,)))
    et = _leaky(s2c + s1r)
    ext = jnp.exp(et - jnp.max(et, axis=0, keepdims=True))
    atT = ext / jnp.sum(ext, axis=0, keepdims=True)
    h_temp = _nn(xb, atT)

    xc = xb[:, _W - _CW:]
    S = _nt(xc, yb) * (1.0 / _CW)
    es = jnp.exp(S - jnp.max(S, axis=1, keepdims=True))
    ac = es / jnp.sum(es, axis=1, keepdims=True)
    ac = ac * (ac > _THRES).astype(jnp.float32)
    agg = _nn(ac, yb)
    h_cause = jnp.maximum(_nn(agg, wc_ref[...]) + bc_ref[...], 0.0)

    hcat = jnp.concatenate([xb, h_feat, h_temp, h_cause], axis=1)
    out_ref[...] = hcat[:, None, None, :].astype(jnp.bfloat16)


def _head_body(hc_ref, wir_ref, wiz_ref, win_ref, whh_ref,
               bir_ref, biz_ref, bin_ref, bhr_ref, bhz_ref, bhn_ref,
               wf1_ref, bf1_ref, wf2_ref, bf2_ref,
               rwir_ref, rwiz_ref, rwin_ref, rwhh_ref,
               rbi_ref, rbh_ref, wo_ref, bo_ref,
               pred_ref, rec_ref, outs_ref, gir_ref, giz_ref, gin_ref):
    whh = whh_ref[...]
    bhn = bhn_ref[...]

    hc2 = hc_ref[...]
    gir_ref[...] = _nn(hc2, wir_ref[...]) + (bir_ref[...] + bhr_ref[...])
    giz_ref[...] = _nn(hc2, wiz_ref[...]) + (biz_ref[...] + bhz_ref[...])
    gin_ref[...] = _nn(hc2, win_ref[...]) + bin_ref[...]

    def enc_step(n, hs):
        hs = list(hs)
        o = n * _B
        gr = [gir_ref[pl.ds(o, 8)], gir_ref[pl.ds(o + 8, 8)]]
        gz = [giz_ref[pl.ds(o, 8)], giz_ref[pl.ds(o + 8, 8)]]
        gn = [gin_ref[pl.ds(o, 8)], gin_ref[pl.ds(o + 8, 8)]]
        for c in range(4):
            half, s = c // 2, slice((c % 2) * 4, (c % 2) * 4 + 4)
            h = hs[c]
            gh = _nn(h.astype(jnp.bfloat16), whh)
            r = _sig(gr[half][s] + gh[:, :_H])
            z = _sig(gz[half][s] + gh[:, _H:2 * _H])
            g = jnp.tanh(gn[half][s] + r * (gh[:, 2 * _H:] + bhn))
            hs[c] = (1.0 - z) * g + z * h
        return tuple(hs)

    z4 = jnp.zeros((4, _H), jnp.float32)
    hs_end = lax.fori_loop(0, _N, enc_step, (z4, z4, z4, z4))
    h_end = jnp.concatenate(hs_end, axis=0)

    f1 = jnp.maximum(_nn(h_end, wf1_ref[...]) + bf1_ref[...], 0.0)
    pred_ref[...] = _nn(f1, wf2_ref[...]) + bf2_ref[...]

    rbi = rbi_ref[...]
    gir = jnp.concatenate(
        [_nn(h_end, rwir_ref[...]), _nn(h_end, rwiz_ref[...]),
         _nn(h_end, rwin_ref[...])], axis=1) + rbi
    rbh = rbh_ref[...]
    gi_r = gir[:, :_RH] + rbh[:, :_RH]
    gi_z = gir[:, _RH:2 * _RH] + rbh[:, _RH:2 * _RH]
    gi_n = gir[:, 2 * _RH:]
    rbhn = rbh[:, 2 * _RH:]
    rwhh = rwhh_ref[...]

    gic_r = [gi_r[c * 4:(c + 1) * 4] for c in range(4)]
    gic_z = [gi_z[c * 4:(c + 1) * 4] for c in range(4)]
    gic_n = [gi_n[c * 4:(c + 1) * 4] for c in range(4)]

    def dec_step(t, hs):
        hs = list(hs)
        for c in range(4):
            h = hs[c]
            gh = _nn(h.astype(jnp.bfloat16), rwhh)
            r = _sig(gic_r[c] + gh[:, :_RH])
            z = _sig(gic_z[c] + gh[:, _RH:2 * _RH])
            g = jnp.tanh(gic_n[c] + r * (gh[:, 2 * _RH:] + rbhn))
            hs[c] = (1.0 - z) * g + z * h
        outs_ref[t] = jnp.concatenate(hs, axis=0)
        return tuple(hs)

    zr = jnp.zeros((4, _RH), jnp.float32)
    lax.fori_loop(0, _W, dec_step, (zr, zr, zr, zr))

    wo = wo_ref[...]
    bo = bo_ref[...]
    for b in range(_B):
        rec_ref[b] = _nn(outs_ref[:, b, :], wo) + bo


def kernel(x, y, emb, Wx_feat, a_feat, W_featlin, b_featlin, w1_temp, w2_temp,
           Wc, bc, W_ih, W_hh, b_ih, b_hh, Wf1, bf1, Wf2, bf2,
           W_ih_r, W_hh_r, b_ih_r, b_hh_r, Wo, bo):
    f32 = jnp.float32
    bf16 = jnp.bfloat16

    sel, wsel, cos = pl.pallas_call(
        _graph_body,
        grid=(1,),
        in_specs=[
            pl.BlockSpec((_N, _E), lambda i: (0, 0)),
            pl.BlockSpec((_E, _N), lambda i: (0, 0)),
            pl.BlockSpec(memory_space=pltpu.SMEM),
        ],
        out_specs=[pl.BlockSpec((_N, _N), lambda i: (0, 0))] * 3,
        out_shape=[jax.ShapeDtypeStruct((_N, _N), f32)] * 3,
    )(emb, emb.T, W_featlin)

    hcat = pl.pallas_call(
        _branch_body,
        grid=(_B,),
        in_specs=[
            pl.BlockSpec((1, _N, _W), lambda b: (b, 0, 0)),
            pl.BlockSpec((1, _N, _CW), lambda b: (b, 0, 0)),
            pl.BlockSpec((_N, _E), lambda b: (0, 0)),
            pl.BlockSpec((_W, _E), lambda b: (0, 0)),
            pl.BlockSpec((_E, 1), lambda b: (0, 0)),
            pl.BlockSpec((_E, 1), lambda b: (0, 0)),
            pl.BlockSpec(memory_space=pltpu.SMEM),
            pl.BlockSpec((_N, 1), lambda b: (0, 0)),
            pl.BlockSpec((_N, 1), lambda b: (0, 0)),
            pl.BlockSpec((_CW, _CH), lambda b: (0, 0)),
            pl.BlockSpec((1, _CH), lambda b: (0, 0)),
            pl.BlockSpec((_N, _N), lambda b: (0, 0)),
            pl.BlockSpec((_N, _N), lambda b: (0, 0)),
            pl.BlockSpec((_N, _N), lambda b: (0, 0)),
        ],
        out_specs=pl.BlockSpec((_N, 1, 1, _DIN), lambda b: (0, b, 0, 0)),
        out_shape=jax.ShapeDtypeStruct((_N, _B, 1, _DIN), jnp.bfloat16),
    )(x, y, emb, Wx_feat,
      a_feat[:_E].reshape(_E, 1), a_feat[_E:].reshape(_E, 1),
      b_featlin.reshape(1, 1),
      w1_temp.reshape(_N, 1), w2_temp.reshape(_N, 1),
      Wc, bc.reshape(1, _CH), sel, wsel, cos)
    hcat = hcat.reshape(_N * _B, _DIN)

    full = lambda s: pl.BlockSpec(s, lambda i: tuple(0 for _ in s))
    pred, rec = pl.pallas_call(
        _head_body,
        grid=(1,),
        in_specs=[
            full((_N * _B, _DIN)),
            full((_DIN, _H)), full((_DIN, _H)), full((_DIN, _H)),
            full((_H, 3 * _H)),
            full((1, _H)), full((1, _H)), full((1, _H)),
            full((1, _H)), full((1, _H)), full((1, _H)),
            full((_H, _FH)), full((1, _FH)), full((_FH, _OUT)), full((1, _OUT)),
            full((_H, _RH)), full((_H, _RH)), full((_H, _RH)),
            full((_RH, 3 * _RH)),
            full((1, 3 * _RH)), full((1, 3 * _RH)),
            full((_RH, _OUT)), full((1, _OUT)),
        ],
        out_specs=[
            full((_B, _OUT)),
            full((_B, _W, _OUT)),
        ],
        out_shape=[
            jax.ShapeDtypeStruct((_B, _OUT), f32),
            jax.ShapeDtypeStruct((_B, _W, _OUT), f32),
        ],
        scratch_shapes=[pltpu.VMEM((_W, _B, _RH), f32),
                        pltpu.VMEM((_N * _B, _H), f32),
                        pltpu.VMEM((_N * _B, _H), f32),
                        pltpu.VMEM((_N * _B, _H), f32)],
    )(hcat,
      W_ih[:, :_H].astype(bf16), W_ih[:, _H:2 * _H].astype(bf16),
      W_ih[:, 2 * _H:].astype(bf16),
      W_hh.astype(bf16),
      b_ih[:_H].reshape(1, _H), b_ih[_H:2 * _H].reshape(1, _H),
      b_ih[2 * _H:].reshape(1, _H),
      b_hh[:_H].reshape(1, _H), b_hh[_H:2 * _H].reshape(1, _H),
      b_hh[2 * _H:].reshape(1, _H),
      Wf1, bf1.reshape(1, _FH), Wf2, bf2.reshape(1, _OUT),
      W_ih_r[:, :_RH], W_ih_r[:, _RH:2 * _RH], W_ih_r[:, 2 * _RH:],
      W_hh_r.astype(bf16),
      b_ih_r.reshape(1, 3 * _RH), b_hh_r.reshape(1, 3 * _RH),
      Wo, bo.reshape(1, _OUT))
    return pred, rec


# speedup vs baseline: 1.1923x; 1.1923x over previous
import jax
import jax.numpy as jnp
from jax import lax
from jax.experimental import pallas as pl
from jax.experimental.pallas import tpu as pltpu

_B, _N, _W, _CW = 16, 256, 100, 10
_E, _CH, _H, _TOPK = 64, 64, 150, 30
_FH, _RH, _OUT = 150, 150, 256
_ALPHA = 0.2
_THRES = 0.1
_DIN = 2 * _W + _E + _CH


def _dot(a, b, dims):
    return lax.dot_general(a, b, (dims, ((), ())),
                           preferred_element_type=jnp.float32)


def _nn(a, b):
    return _dot(a, b, ((1,), (0,)))


def _nt(a, b):
    return _dot(a, b, ((1,), (1,)))


def _leaky(v):
    return jnp.where(v > 0, v, _ALPHA * v)


def _sig(v):
    return 0.5 * jnp.tanh(0.5 * v) + 0.5


def _graph_body(emb_ref, embt_ref, wfl_ref, sel_ref, wsel_ref, cos_ref):
    emb = emb_ref[...]
    embt = embt_ref[...]
    ncol = jnp.sqrt(jnp.sum(emb * emb, axis=1, keepdims=True))
    nrow = jnp.sqrt(jnp.sum(embt * embt, axis=0, keepdims=True))
    cos = _nt(emb, emb) / (ncol * nrow)
    ri = lax.broadcasted_iota(jnp.int32, (_N, _N), 0)
    ci = lax.broadcasted_iota(jnp.int32, (_N, _N), 1)
    cos = jnp.where(ri != ci, cos, 0.0)
    cos_ref[...] = cos

    work = cos
    sel = jnp.zeros((_N, _N), jnp.float32)
    wsel = jnp.zeros((_N, _N), jnp.float32)
    for k in range(_TOPK):
        m = jnp.max(work, axis=1, keepdims=True)
        jmin = jnp.min(jnp.where(work == m, ci, _N), axis=1, keepdims=True)
        oh = (ci == jmin).astype(jnp.float32)
        sel = sel + oh
        wsel = wsel + oh * wfl_ref[k, 0]
        work = work - oh * jnp.float32(1e30)
    sel_ref[...] = sel
    wsel_ref[...] = wsel


def _branch_body(x_ref, y_ref, emb_ref, wx_ref, a1_ref, a2_ref, bfl_ref,
                 w1_ref, w2_ref, wc_ref, bc_ref, sel_ref, wsel_ref, cos_ref,
                 out_ref):
    xb = x_ref[0]
    yb = y_ref[0]

    nr = _nn(xb, wx_ref[...]) + emb_ref[...]
    d1 = _nn(nr, a1_ref[...])
    d2r = _dot(a2_ref[...], nr, ((0,), (1,)))
    e = _leaky(d1 + d2r) + cos_ref[...]
    sel = sel_ref[...]
    em = jnp.where(sel > 0, e, -1e30)
    ex = jnp.exp(em - jnp.max(em, axis=1, keepdims=True)) * sel
    aw = ex / jnp.sum(ex, axis=1, keepdims=True) * wsel_ref[...]
    h_feat = _nn(aw, nr) + bfl_ref[0, 0]

    s1r = _dot(w1_ref[...], xb, ((0,), (0,)))
    s2c = _dot(xb, w2_ref[...], ((0,), (0,)))
    et = _leaky(s2c + s1r)
    ext = jnp.exp(et - jnp.max(et, axis=0, keepdims=True))
    atT = ext / jnp.sum(ext, axis=0, keepdims=True)
    h_temp = _nn(xb, atT)

    xc = xb[:, _W - _CW:]
    S = _nt(xc, yb) * (1.0 / _CW)
    es = jnp.exp(S - jnp.max(S, axis=1, keepdims=True))
    ac = es / jnp.sum(es, axis=1, keepdims=True)
    ac = ac * (ac > _THRES).astype(jnp.float32)
    agg = _nn(ac, yb)
    h_cause = jnp.maximum(_nn(agg, wc_ref[...]) + bc_ref[...], 0.0)

    hcat = jnp.concatenate([xb, h_feat, h_temp, h_cause], axis=1)
    out_ref[...] = hcat[:, None, None, :].astype(jnp.bfloat16)


def _head_body(hc_ref, wir_ref, wiz_ref, win_ref, whh_ref,
               bir_ref, biz_ref, bin_ref, bhr_ref, bhz_ref, bhn_ref,
               wf1_ref, bf1_ref, wf2_ref, bf2_ref,
               rwir_ref, rwiz_ref, rwin_ref, rwhh_ref,
               rbi_ref, rbh_ref, wo_ref, bo_ref,
               pred_ref, rec_ref, outs_ref, gir_ref, giz_ref, gin_ref):
    whh = whh_ref[...]
    bhn = bhn_ref[...]

    hc2 = hc_ref[...]
    gir_ref[...] = _nn(hc2, wir_ref[...]) + (bir_ref[...] + bhr_ref[...])
    giz_ref[...] = _nn(hc2, wiz_ref[...]) + (biz_ref[...] + bhz_ref[...])
    gin_ref[...] = _nn(hc2, win_ref[...]) + bin_ref[...]

    def enc_block(i, h):
        for j in range(4):
            o = (i * 4 + j) * _B
            gh = _nn(h.astype(jnp.bfloat16), whh)
            r = _sig(gir_ref[pl.ds(o, _B)] + gh[:, :_H])
            z = _sig(giz_ref[pl.ds(o, _B)] + gh[:, _H:2 * _H])
            g = jnp.tanh(gin_ref[pl.ds(o, _B)] + r * (gh[:, 2 * _H:] + bhn))
            h = (1.0 - z) * g + z * h
        return h

    h_end = lax.fori_loop(0, _N // 4, enc_block,
                          jnp.zeros((_B, _H), jnp.float32))

    f1 = jnp.maximum(_nn(h_end, wf1_ref[...]) + bf1_ref[...], 0.0)
    pred_ref[...] = _nn(f1, wf2_ref[...]) + bf2_ref[...]

    rbi = rbi_ref[...]
    gir = jnp.concatenate(
        [_nn(h_end, rwir_ref[...]), _nn(h_end, rwiz_ref[...]),
         _nn(h_end, rwin_ref[...])], axis=1) + rbi
    rbh = rbh_ref[...]
    gi_r = gir[:, :_RH] + rbh[:, :_RH]
    gi_z = gir[:, _RH:2 * _RH] + rbh[:, _RH:2 * _RH]
    gi_n = gir[:, 2 * _RH:]
    rbhn = rbh[:, 2 * _RH:]
    rwhh = rwhh_ref[...]

    def dec_block(i, h):
        for j in range(4):
            t = i * 4 + j
            gh = _nn(h.astype(jnp.bfloat16), rwhh)
            r = _sig(gi_r + gh[:, :_RH])
            z = _sig(gi_z + gh[:, _RH:2 * _RH])
            g = jnp.tanh(gi_n + r * (gh[:, 2 * _RH:] + rbhn))
            h = (1.0 - z) * g + z * h
            outs_ref[t] = h
        return h

    lax.fori_loop(0, _W // 4, dec_block, jnp.zeros((_B, _RH), jnp.float32))

    wo = wo_ref[...]
    bo = bo_ref[...]
    for b in range(_B):
        rec_ref[b] = _nn(outs_ref[:, b, :], wo) + bo


def kernel(x, y, emb, Wx_feat, a_feat, W_featlin, b_featlin, w1_temp, w2_temp,
           Wc, bc, W_ih, W_hh, b_ih, b_hh, Wf1, bf1, Wf2, bf2,
           W_ih_r, W_hh_r, b_ih_r, b_hh_r, Wo, bo):
    f32 = jnp.float32
    bf16 = jnp.bfloat16

    sel, wsel, cos = pl.pallas_call(
        _graph_body,
        grid=(1,),
        in_specs=[
            pl.BlockSpec((_N, _E), lambda i: (0, 0)),
            pl.BlockSpec((_E, _N), lambda i: (0, 0)),
            pl.BlockSpec(memory_space=pltpu.SMEM),
        ],
        out_specs=[pl.BlockSpec((_N, _N), lambda i: (0, 0))] * 3,
        out_shape=[jax.ShapeDtypeStruct((_N, _N), f32)] * 3,
    )(emb, emb.T, W_featlin)

    hcat = pl.pallas_call(
        _branch_body,
        grid=(_B,),
        in_specs=[
            pl.BlockSpec((1, _N, _W), lambda b: (b, 0, 0)),
            pl.BlockSpec((1, _N, _CW), lambda b: (b, 0, 0)),
            pl.BlockSpec((_N, _E), lambda b: (0, 0)),
            pl.BlockSpec((_W, _E), lambda b: (0, 0)),
            pl.BlockSpec((_E, 1), lambda b: (0, 0)),
            pl.BlockSpec((_E, 1), lambda b: (0, 0)),
            pl.BlockSpec(memory_space=pltpu.SMEM),
            pl.BlockSpec((_N, 1), lambda b: (0, 0)),
            pl.BlockSpec((_N, 1), lambda b: (0, 0)),
            pl.BlockSpec((_CW, _CH), lambda b: (0, 0)),
            pl.BlockSpec((1, _CH), lambda b: (0, 0)),
            pl.BlockSpec((_N, _N), lambda b: (0, 0)),
            pl.BlockSpec((_N, _N), lambda b: (0, 0)),
            pl.BlockSpec((_N, _N), lambda b: (0, 0)),
        ],
        out_specs=pl.BlockSpec((_N, 1, 1, _DIN), lambda b: (0, b, 0, 0)),
        out_shape=jax.ShapeDtypeStruct((_N, _B, 1, _DIN), jnp.bfloat16),
    )(x, y, emb, Wx_feat,
      a_feat[:_E].reshape(_E, 1), a_feat[_E:].reshape(_E, 1),
      b_featlin.reshape(1, 1),
      w1_temp.reshape(_N, 1), w2_temp.reshape(_N, 1),
      Wc, bc.reshape(1, _CH), sel, wsel, cos)
    hcat = hcat.reshape(_N * _B, _DIN)

    full = lambda s: pl.BlockSpec(s, lambda i: tuple(0 for _ in s))
    pred, rec = pl.pallas_call(
        _head_body,
        grid=(1,),
        in_specs=[
            full((_N * _B, _DIN)),
            full((_DIN, _H)), full((_DIN, _H)), full((_DIN, _H)),
            full((_H, 3 * _H)),
            full((1, _H)), full((1, _H)), full((1, _H)),
            full((1, _H)), full((1, _H)), full((1, _H)),
            full((_H, _FH)), full((1, _FH)), full((_FH, _OUT)), full((1, _OUT)),
            full((_H, _RH)), full((_H, _RH)), full((_H, _RH)),
            full((_RH, 3 * _RH)),
            full((1, 3 * _RH)), full((1, 3 * _RH)),
            full((_RH, _OUT)), full((1, _OUT)),
        ],
        out_specs=[
            full((_B, _OUT)),
            full((_B, _W, _OUT)),
        ],
        out_shape=[
            jax.ShapeDtypeStruct((_B, _OUT), f32),
            jax.ShapeDtypeStruct((_B, _W, _OUT), f32),
        ],
        scratch_shapes=[pltpu.VMEM((_W, _B, _RH), f32),
                        pltpu.VMEM((_N * _B, _H), f32),
                        pltpu.VMEM((_N * _B, _H), f32),
                        pltpu.VMEM((_N * _B, _H), f32)],
    )(hcat,
      W_ih[:, :_H].astype(bf16), W_ih[:, _H:2 * _H].astype(bf16),
      W_ih[:, 2 * _H:].astype(bf16),
      W_hh.astype(bf16),
      b_ih[:_H].reshape(1, _H), b_ih[_H:2 * _H].reshape(1, _H),
      b_ih[2 * _H:].reshape(1, _H),
      b_hh[:_H].reshape(1, _H), b_hh[_H:2 * _H].reshape(1, _H),
      b_hh[2 * _H:].reshape(1, _H),
      Wf1, bf1.reshape(1, _FH), Wf2, bf2.reshape(1, _OUT),
      W_ih_r[:, :_RH], W_ih_r[:, _RH:2 * _RH], W_ih_r[:, 2 * _RH:],
      W_hh_r.astype(bf16),
      b_ih_r.reshape(1, 3 * _RH), b_hh_r.reshape(1, 3 * _RH),
      Wo, bo.reshape(1, _OUT))
    return pred, rec


# speedup vs baseline: 1.2165x; 1.0203x over previous
import jax
import jax.numpy as jnp
from jax import lax
from jax.experimental import pallas as pl
from jax.experimental.pallas import tpu as pltpu

_B, _N, _W, _CW = 16, 256, 100, 10
_E, _CH, _H, _TOPK = 64, 64, 150, 30
_FH, _RH, _OUT = 150, 150, 256
_ALPHA = 0.2
_THRES = 0.1
_DIN = 2 * _W + _E + _CH


def _dot(a, b, dims):
    return lax.dot_general(a, b, (dims, ((), ())),
                           preferred_element_type=jnp.float32)


def _nn(a, b):
    return _dot(a, b, ((1,), (0,)))


def _nt(a, b):
    return _dot(a, b, ((1,), (1,)))


def _leaky(v):
    return jnp.where(v > 0, v, _ALPHA * v)


def _sig(v):
    return 0.5 * jnp.tanh(0.5 * v) + 0.5


def _branch_body(x_ref, y_ref, emb_ref, embt_ref, wfl_ref, wx_ref, a1_ref,
                 a2_ref, bfl_ref, w1_ref, w2_ref, wc_ref, bc_ref,
                 out_ref, sel_ref, wsel_ref, cos_ref):
    @pl.when(pl.program_id(0) == 0)
    def _graph():
        emb = emb_ref[...]
        embt = embt_ref[...]
        ncol = jnp.sqrt(jnp.sum(emb * emb, axis=1, keepdims=True))
        nrow = jnp.sqrt(jnp.sum(embt * embt, axis=0, keepdims=True))
        cos = _nt(emb, emb) / (ncol * nrow)
        ri = lax.broadcasted_iota(jnp.int32, (_N, _N), 0)
        ci = lax.broadcasted_iota(jnp.int32, (_N, _N), 1)
        cos = jnp.where(ri != ci, cos, 0.0)
        cos_ref[...] = cos

        work = cos
        sel = jnp.zeros((_N, _N), jnp.float32)
        wsel = jnp.zeros((_N, _N), jnp.float32)
        for k in range(_TOPK):
            m = jnp.max(work, axis=1, keepdims=True)
            jmin = jnp.min(jnp.where(work == m, ci, _N), axis=1, keepdims=True)
            oh = (ci == jmin).astype(jnp.float32)
            sel = sel + oh
            wsel = wsel + oh * wfl_ref[k, 0]
            work = work - oh * jnp.float32(1e30)
        sel_ref[...] = sel
        wsel_ref[...] = wsel

    xb = x_ref[0]
    yb = y_ref[0]

    nr = _nn(xb, wx_ref[...]) + emb_ref[...]
    d1 = _nn(nr, a1_ref[...])
    d2r = _dot(a2_ref[...], nr, ((0,), (1,)))
    e = _leaky(d1 + d2r) + cos_ref[...]
    sel = sel_ref[...]
    em = jnp.where(sel > 0, e, -1e30)
    ex = jnp.exp(em - jnp.max(em, axis=1, keepdims=True)) * sel
    aw = ex / jnp.sum(ex, axis=1, keepdims=True) * wsel_ref[...]
    h_feat = _nn(aw, nr) + bfl_ref[0, 0]

    s1r = _dot(w1_ref[...], xb, ((0,), (0,)))
    s2c = _dot(xb, w2_ref[...], ((0,), (0,)))
    et = _leaky(s2c + s1r)
    ext = jnp.exp(et - jnp.max(et, axis=0, keepdims=True))
    atT = ext / jnp.sum(ext, axis=0, keepdims=True)
    h_temp = _nn(xb, atT)

    xc = xb[:, _W - _CW:]
    S = _nt(xc, yb) * (1.0 / _CW)
    es = jnp.exp(S - jnp.max(S, axis=1, keepdims=True))
    ac = es / jnp.sum(es, axis=1, keepdims=True)
    ac = ac * (ac > _THRES).astype(jnp.float32)
    agg = _nn(ac, yb)
    h_cause = jnp.maximum(_nn(agg, wc_ref[...]) + bc_ref[...], 0.0)

    hcat = jnp.concatenate([xb, h_feat, h_temp, h_cause], axis=1)
    out_ref[...] = hcat[:, None, None, :].astype(jnp.bfloat16)


def _head_body(hc_ref, wir_ref, wiz_ref, win_ref, whh_ref,
               bir_ref, biz_ref, bin_ref, bhr_ref, bhz_ref, bhn_ref,
               wf1_ref, bf1_ref, wf2_ref, bf2_ref,
               rwir_ref, rwiz_ref, rwin_ref, rwhh_ref,
               rbi_ref, rbh_ref, wo_ref, bo_ref,
               pred_ref, rec_ref, outs_ref, gir_ref, giz_ref, gin_ref):
    whh = whh_ref[...]
    bhn = bhn_ref[...]

    hc2 = hc_ref[...]
    gir_ref[...] = _nn(hc2, wir_ref[...]) + (bir_ref[...] + bhr_ref[...])
    giz_ref[...] = _nn(hc2, wiz_ref[...]) + (biz_ref[...] + bhz_ref[...])
    gin_ref[...] = _nn(hc2, win_ref[...]) + bin_ref[...]

    def enc_block(i, h):
        for j in range(4):
            o = (i * 4 + j) * _B
            gh = _nn(h.astype(jnp.bfloat16), whh)
            r = _sig(gir_ref[pl.ds(o, _B)] + gh[:, :_H])
            z = _sig(giz_ref[pl.ds(o, _B)] + gh[:, _H:2 * _H])
            g = jnp.tanh(gin_ref[pl.ds(o, _B)] + r * (gh[:, 2 * _H:] + bhn))
            h = (1.0 - z) * g + z * h
        return h

    h_end = lax.fori_loop(0, _N // 4, enc_block,
                          jnp.zeros((_B, _H), jnp.float32))

    f1 = jnp.maximum(_nn(h_end, wf1_ref[...]) + bf1_ref[...], 0.0)
    pred_ref[...] = _nn(f1, wf2_ref[...]) + bf2_ref[...]

    rbi = rbi_ref[...]
    gir = jnp.concatenate(
        [_nn(h_end, rwir_ref[...]), _nn(h_end, rwiz_ref[...]),
         _nn(h_end, rwin_ref[...])], axis=1) + rbi
    rbh = rbh_ref[...]
    gi_r = gir[:, :_RH] + rbh[:, :_RH]
    gi_z = gir[:, _RH:2 * _RH] + rbh[:, _RH:2 * _RH]
    gi_n = gir[:, 2 * _RH:]
    rbhn = rbh[:, 2 * _RH:]
    rwhh = rwhh_ref[...]

    def dec_block(i, h):
        for j in range(4):
            t = i * 4 + j
            gh = _nn(h.astype(jnp.bfloat16), rwhh)
            r = _sig(gi_r + gh[:, :_RH])
            z = _sig(gi_z + gh[:, _RH:2 * _RH])
            g = jnp.tanh(gi_n + r * (gh[:, 2 * _RH:] + rbhn))
            h = (1.0 - z) * g + z * h
            outs_ref[t] = h
        return h

    lax.fori_loop(0, _W // 4, dec_block, jnp.zeros((_B, _RH), jnp.float32))

    wo = wo_ref[...]
    bo = bo_ref[...]
    for b in range(_B):
        rec_ref[b] = _nn(outs_ref[:, b, :], wo) + bo


def kernel(x, y, emb, Wx_feat, a_feat, W_featlin, b_featlin, w1_temp, w2_temp,
           Wc, bc, W_ih, W_hh, b_ih, b_hh, Wf1, bf1, Wf2, bf2,
           W_ih_r, W_hh_r, b_ih_r, b_hh_r, Wo, bo):
    f32 = jnp.float32
    bf16 = jnp.bfloat16

    hcat = pl.pallas_call(
        _branch_body,
        grid=(_B,),
        in_specs=[
            pl.BlockSpec((1, _N, _W), lambda b: (b, 0, 0)),
            pl.BlockSpec((1, _N, _CW), lambda b: (b, 0, 0)),
            pl.BlockSpec((_N, _E), lambda b: (0, 0)),
            pl.BlockSpec((_E, _N), lambda b: (0, 0)),
            pl.BlockSpec(memory_space=pltpu.SMEM),
            pl.BlockSpec((_W, _E), lambda b: (0, 0)),
            pl.BlockSpec((_E, 1), lambda b: (0, 0)),
            pl.BlockSpec((_E, 1), lambda b: (0, 0)),
            pl.BlockSpec(memory_space=pltpu.SMEM),
            pl.BlockSpec((_N, 1), lambda b: (0, 0)),
            pl.BlockSpec((_N, 1), lambda b: (0, 0)),
            pl.BlockSpec((_CW, _CH), lambda b: (0, 0)),
            pl.BlockSpec((1, _CH), lambda b: (0, 0)),
        ],
        out_specs=pl.BlockSpec((_N, 1, 1, _DIN), lambda b: (0, b, 0, 0)),
        out_shape=jax.ShapeDtypeStruct((_N, _B, 1, _DIN), jnp.bfloat16),
        scratch_shapes=[pltpu.VMEM((_N, _N), f32)] * 3,
    )(x, y, emb, emb.T, W_featlin, Wx_feat,
      a_feat[:_E].reshape(_E, 1), a_feat[_E:].reshape(_E, 1),
      b_featlin.reshape(1, 1),
      w1_temp.reshape(_N, 1), w2_temp.reshape(_N, 1),
      Wc, bc.reshape(1, _CH))
    hcat = hcat.reshape(_N * _B, _DIN)

    full = lambda s: pl.BlockSpec(s, lambda i: tuple(0 for _ in s))
    pred, rec = pl.pallas_call(
        _head_body,
        grid=(1,),
        in_specs=[
            full((_N * _B, _DIN)),
            full((_DIN, _H)), full((_DIN, _H)), full((_DIN, _H)),
            full((_H, 3 * _H)),
            full((1, _H)), full((1, _H)), full((1, _H)),
            full((1, _H)), full((1, _H)), full((1, _H)),
            full((_H, _FH)), full((1, _FH)), full((_FH, _OUT)), full((1, _OUT)),
            full((_H, _RH)), full((_H, _RH)), full((_H, _RH)),
            full((_RH, 3 * _RH)),
            full((1, 3 * _RH)), full((1, 3 * _RH)),
            full((_RH, _OUT)), full((1, _OUT)),
        ],
        out_specs=[
            full((_B, _OUT)),
            full((_B, _W, _OUT)),
        ],
        out_shape=[
            jax.ShapeDtypeStruct((_B, _OUT), f32),
            jax.ShapeDtypeStruct((_B, _W, _OUT), f32),
        ],
        scratch_shapes=[pltpu.VMEM((_W, _B, _RH), f32),
                        pltpu.VMEM((_N * _B, _H), f32),
                        pltpu.VMEM((_N * _B, _H), f32),
                        pltpu.VMEM((_N * _B, _H), f32)],
    )(hcat,
      W_ih[:, :_H].astype(bf16), W_ih[:, _H:2 * _H].astype(bf16),
      W_ih[:, 2 * _H:].astype(bf16),
      W_hh.astype(bf16),
      b_ih[:_H].reshape(1, _H), b_ih[_H:2 * _H].reshape(1, _H),
      b_ih[2 * _H:].reshape(1, _H),
      b_hh[:_H].reshape(1, _H), b_hh[_H:2 * _H].reshape(1, _H),
      b_hh[2 * _H:].reshape(1, _H),
      Wf1, bf1.reshape(1, _FH), Wf2, bf2.reshape(1, _OUT),
      W_ih_r[:, :_RH], W_ih_r[:, _RH:2 * _RH], W_ih_r[:, 2 * _RH:],
      W_hh_r.astype(bf16),
      b_ih_r.reshape(1, 3 * _RH), b_hh_r.reshape(1, 3 * _RH),
      Wo, bo.reshape(1, _OUT))
    return pred, rec


# speedup vs baseline: 1.2241x; 1.0063x over previous
import jax
import jax.numpy as jnp
from jax import lax
from jax.experimental import pallas as pl
from jax.experimental.pallas import tpu as pltpu

_B, _N, _W, _CW = 16, 256, 100, 10
_E, _CH, _H, _TOPK = 64, 64, 150, 30
_FH, _RH, _OUT = 150, 150, 256
_ALPHA = 0.2
_THRES = 0.1
_DIN = 2 * _W + _E + _CH


def _dot(a, b, dims):
    return lax.dot_general(a, b, (dims, ((), ())),
                           preferred_element_type=jnp.float32)


def _nn(a, b):
    return _dot(a, b, ((1,), (0,)))


def _nt(a, b):
    return _dot(a, b, ((1,), (1,)))


def _leaky(v):
    return jnp.where(v > 0, v, _ALPHA * v)


def _sig(v):
    return 0.5 * jnp.tanh(0.5 * v) + 0.5


def _branch_body(x_ref, y_ref, emb_ref, embt_ref, wfl_ref, wx_ref, a1_ref,
                 a2_ref, bfl_ref, w1_ref, w2_ref, wc_ref, bc_ref,
                 out_ref, sel_ref, wsel_ref, cos_ref):
    @pl.when(pl.program_id(0) == 0)
    def _graph():
        emb = emb_ref[...]
        embt = embt_ref[...]
        ncol = jnp.sqrt(jnp.sum(emb * emb, axis=1, keepdims=True))
        nrow = jnp.sqrt(jnp.sum(embt * embt, axis=0, keepdims=True))
        cos = _nt(emb, emb) / (ncol * nrow)
        ri = lax.broadcasted_iota(jnp.int32, (_N, _N), 0)
        ci = lax.broadcasted_iota(jnp.int32, (_N, _N), 1)
        cos = jnp.where(ri != ci, cos, 0.0)
        cos_ref[...] = cos

        work = cos
        sel = jnp.zeros((_N, _N), jnp.float32)
        wsel = jnp.zeros((_N, _N), jnp.float32)
        for k in range(_TOPK):
            m = jnp.max(work, axis=1, keepdims=True)
            jmin = jnp.min(jnp.where(work == m, ci, _N), axis=1, keepdims=True)
            oh = (ci == jmin).astype(jnp.float32)
            sel = sel + oh
            wsel = wsel + oh * wfl_ref[k, 0]
            work = work - oh * jnp.float32(1e30)
        sel_ref[...] = sel
        wsel_ref[...] = wsel

    xb = x_ref[0]
    yb = y_ref[0]

    nr = _nn(xb, wx_ref[...]) + emb_ref[...]
    d1 = _nn(nr, a1_ref[...])
    d2r = _dot(a2_ref[...], nr, ((0,), (1,)))
    e = _leaky(d1 + d2r) + cos_ref[...]
    sel = sel_ref[...]
    em = jnp.where(sel > 0, e, -1e30)
    ex = jnp.exp(em - jnp.max(em, axis=1, keepdims=True)) * sel
    aw = ex / jnp.sum(ex, axis=1, keepdims=True) * wsel_ref[...]
    h_feat = _nn(aw, nr) + bfl_ref[0, 0]

    s1r = _dot(w1_ref[...], xb, ((0,), (0,)))
    s2c = _dot(xb, w2_ref[...], ((0,), (0,)))
    et = _leaky(s2c + s1r)
    ext = jnp.exp(et - jnp.max(et, axis=0, keepdims=True))
    atT = ext / jnp.sum(ext, axis=0, keepdims=True)
    h_temp = _nn(xb, atT)

    xc = xb[:, _W - _CW:]
    S = _nt(xc, yb) * (1.0 / _CW)
    es = jnp.exp(S - jnp.max(S, axis=1, keepdims=True))
    ac = es / jnp.sum(es, axis=1, keepdims=True)
    ac = ac * (ac > _THRES).astype(jnp.float32)
    agg = _nn(ac, yb)
    h_cause = jnp.maximum(_nn(agg, wc_ref[...]) + bc_ref[...], 0.0)

    hcat = jnp.concatenate([xb, h_feat, h_temp, h_cause], axis=1)
    out_ref[...] = hcat[:, None, None, :].astype(jnp.bfloat16)


def _head_body(hc_ref, wir_ref, wiz_ref, win_ref, whh_ref,
               bir_ref, biz_ref, bin_ref, bhr_ref, bhz_ref, bhn_ref,
               wf1_ref, bf1_ref, wf2_ref, bf2_ref,
               rwir_ref, rwiz_ref, rwin_ref, rwhh_ref,
               rbi_ref, rbh_ref, wo_ref, bo_ref,
               pred_ref, rec_ref, outs_ref, gir_ref, giz_ref, gin_ref):
    whh = whh_ref[...]
    bhn = bhn_ref[...]

    hc2 = hc_ref[...]
    gir_ref[...] = _nn(hc2, wir_ref[...]) + (bir_ref[...] + bhr_ref[...])
    giz_ref[...] = _nn(hc2, wiz_ref[...]) + (biz_ref[...] + bhz_ref[...])
    gin_ref[...] = _nn(hc2, win_ref[...]) + bin_ref[...]

    def enc_block(i, h):
        for j in range(8):
            o = (i * 8 + j) * _B
            gh = _nn(h.astype(jnp.bfloat16), whh)
            r = _sig(gir_ref[pl.ds(o, _B)] + gh[:, :_H])
            z = _sig(giz_ref[pl.ds(o, _B)] + gh[:, _H:2 * _H])
            g = jnp.tanh(gin_ref[pl.ds(o, _B)] + r * (gh[:, 2 * _H:] + bhn))
            h = (1.0 - z) * g + z * h
        return h

    h_end = lax.fori_loop(0, _N // 8, enc_block,
                          jnp.zeros((_B, _H), jnp.float32))

    f1 = jnp.maximum(_nn(h_end, wf1_ref[...]) + bf1_ref[...], 0.0)
    pred_ref[...] = _nn(f1, wf2_ref[...]) + bf2_ref[...]

    rbi = rbi_ref[...]
    gir = jnp.concatenate(
        [_nn(h_end, rwir_ref[...]), _nn(h_end, rwiz_ref[...]),
         _nn(h_end, rwin_ref[...])], axis=1) + rbi
    rbh = rbh_ref[...]
    gi_r = gir[:, :_RH] + rbh[:, :_RH]
    gi_z = gir[:, _RH:2 * _RH] + rbh[:, _RH:2 * _RH]
    gi_n = gir[:, 2 * _RH:]
    rbhn = rbh[:, 2 * _RH:]
    rwhh = rwhh_ref[...]

    def dec_block(i, h):
        for j in range(4):
            t = i * 4 + j
            gh = _nn(h.astype(jnp.bfloat16), rwhh)
            r = _sig(gi_r + gh[:, :_RH])
            z = _sig(gi_z + gh[:, _RH:2 * _RH])
            g = jnp.tanh(gi_n + r * (gh[:, 2 * _RH:] + rbhn))
            h = (1.0 - z) * g + z * h
            outs_ref[t] = h
        return h

    lax.fori_loop(0, _W // 4, dec_block, jnp.zeros((_B, _RH), jnp.float32))

    wo = wo_ref[...]
    bo = bo_ref[...]
    for b in range(_B):
        rec_ref[b] = _nn(outs_ref[:, b, :], wo) + bo


def kernel(x, y, emb, Wx_feat, a_feat, W_featlin, b_featlin, w1_temp, w2_temp,
           Wc, bc, W_ih, W_hh, b_ih, b_hh, Wf1, bf1, Wf2, bf2,
           W_ih_r, W_hh_r, b_ih_r, b_hh_r, Wo, bo):
    f32 = jnp.float32
    bf16 = jnp.bfloat16

    hcat = pl.pallas_call(
        _branch_body,
        grid=(_B,),
        in_specs=[
            pl.BlockSpec((1, _N, _W), lambda b: (b, 0, 0)),
            pl.BlockSpec((1, _N, _CW), lambda b: (b, 0, 0)),
            pl.BlockSpec((_N, _E), lambda b: (0, 0)),
            pl.BlockSpec((_E, _N), lambda b: (0, 0)),
            pl.BlockSpec(memory_space=pltpu.SMEM),
            pl.BlockSpec((_W, _E), lambda b: (0, 0)),
            pl.BlockSpec((_E, 1), lambda b: (0, 0)),
            pl.BlockSpec((_E, 1), lambda b: (0, 0)),
            pl.BlockSpec(memory_space=pltpu.SMEM),
            pl.BlockSpec((_N, 1), lambda b: (0, 0)),
            pl.BlockSpec((_N, 1), lambda b: (0, 0)),
            pl.BlockSpec((_CW, _CH), lambda b: (0, 0)),
            pl.BlockSpec((1, _CH), lambda b: (0, 0)),
        ],
        out_specs=pl.BlockSpec((_N, 1, 1, _DIN), lambda b: (0, b, 0, 0)),
        out_shape=jax.ShapeDtypeStruct((_N, _B, 1, _DIN), jnp.bfloat16),
        scratch_shapes=[pltpu.VMEM((_N, _N), f32)] * 3,
    )(x, y, emb, emb.T, W_featlin, Wx_feat,
      a_feat[:_E].reshape(_E, 1), a_feat[_E:].reshape(_E, 1),
      b_featlin.reshape(1, 1),
      w1_temp.reshape(_N, 1), w2_temp.reshape(_N, 1),
      Wc, bc.reshape(1, _CH))
    hcat = hcat.reshape(_N * _B, _DIN)

    full = lambda s: pl.BlockSpec(s, lambda i: tuple(0 for _ in s))
    pred, rec = pl.pallas_call(
        _head_body,
        grid=(1,),
        in_specs=[
            full((_N * _B, _DIN)),
            full((_DIN, _H)), full((_DIN, _H)), full((_DIN, _H)),
            full((_H, 3 * _H)),
            full((1, _H)), full((1, _H)), full((1, _H)),
            full((1, _H)), full((1, _H)), full((1, _H)),
            full((_H, _FH)), full((1, _FH)), full((_FH, _OUT)), full((1, _OUT)),
            full((_H, _RH)), full((_H, _RH)), full((_H, _RH)),
            full((_RH, 3 * _RH)),
            full((1, 3 * _RH)), full((1, 3 * _RH)),
            full((_RH, _OUT)), full((1, _OUT)),
        ],
        out_specs=[
            full((_B, _OUT)),
            full((_B, _W, _OUT)),
        ],
        out_shape=[
            jax.ShapeDtypeStruct((_B, _OUT), f32),
            jax.ShapeDtypeStruct((_B, _W, _OUT), f32),
        ],
        scratch_shapes=[pltpu.VMEM((_W, _B, _RH), f32),
                        pltpu.VMEM((_N * _B, _H), f32),
                        pltpu.VMEM((_N * _B, _H), f32),
                        pltpu.VMEM((_N * _B, _H), f32)],
    )(hcat,
      W_ih[:, :_H].astype(bf16), W_ih[:, _H:2 * _H].astype(bf16),
      W_ih[:, 2 * _H:].astype(bf16),
      W_hh.astype(bf16),
      b_ih[:_H].reshape(1, _H), b_ih[_H:2 * _H].reshape(1, _H),
      b_ih[2 * _H:].reshape(1, _H),
      b_hh[:_H].reshape(1, _H), b_hh[_H:2 * _H].reshape(1, _H),
      b_hh[2 * _H:].reshape(1, _H),
      Wf1, bf1.reshape(1, _FH), Wf2, bf2.reshape(1, _OUT),
      W_ih_r[:, :_RH], W_ih_r[:, _RH:2 * _RH], W_ih_r[:, 2 * _RH:],
      W_hh_r.astype(bf16),
      b_ih_r.reshape(1, 3 * _RH), b_hh_r.reshape(1, 3 * _RH),
      Wo, bo.reshape(1, _OUT))
    return pred, rec
